# baseline (speedup 1.0000x reference)
import jax
import jax.numpy as jnp
from jax.experimental import pallas as pl
from jax.experimental.pallas import tpu as pltpu

C = 100000
B = 1024
SMOOTH = 0.1
CONF = 1.0 - SMOOTH
SVAL = SMOOTH / (C - 1)

BV = 2000
NV = C // BV
NG = BV // 8


def _loss_body(t_ref, x_ref, o_ref, m_ref, s_ref, rs_ref, tg_ref):
    v = pl.program_id(0)
    nv = pl.num_programs(0)

    @pl.when(v == 0)
    def _init():
        m_ref[...] = jnp.full((8, B), -jnp.inf, jnp.float32)
        s_ref[...] = jnp.zeros((8, B), jnp.float32)
        rs_ref[...] = jnp.zeros((8, B), jnp.float32)
        tg_ref[...] = jnp.zeros((8, B), jnp.float32)

    sub = jax.lax.broadcasted_iota(jnp.int32, (8, B), 0)
    trel = t_ref[...] - v * BV + jnp.zeros((8, B), jnp.int32)

    def p1(g, carry):
        bmax, rs = carry
        xg = x_ref[pl.ds(pl.multiple_of(g * 8, 8), 8), :]
        return jnp.maximum(bmax, xg), rs + xg

    bmax, rs = jax.lax.fori_loop(
        0, NG, p1,
        (jnp.full((8, B), -jnp.inf, jnp.float32), rs_ref[...]))
    rs_ref[...] = rs
    m_old = m_ref[...]
    m_new = jnp.maximum(m_old, bmax)
    m_ref[...] = m_new

    def p2(g, carry):
        acc, tg = carry
        xg = x_ref[pl.ds(pl.multiple_of(g * 8, 8), 8), :]
        hit = trel - g * 8 == sub
        return acc + jnp.exp(xg - m_new), tg + jnp.where(hit, xg, 0.0)

    acc, tg = jax.lax.fori_loop(
        0, NG, p2, (s_ref[...] * jnp.exp(m_old - m_new), tg_ref[...]))
    s_ref[...] = acc
    tg_ref[...] = tg

    @pl.when(v == nv - 1)
    def _fin():
        m_acc = m_ref[...]
        m_col = jnp.max(m_acc, axis=0, keepdims=True)
        s_col = jnp.sum(s_ref[...] * jnp.exp(m_acc - m_col), axis=0,
                        keepdims=True)
        lse = m_col + jnp.log(s_col)
        rs_col = jnp.sum(rs_ref[...], axis=0, keepdims=True)
        g_col = jnp.sum(tg_ref[...], axis=0, keepdims=True)
        loss = -(SVAL * (rs_col - C * lse) + (CONF - SVAL) * (g_col - lse))
        o_ref[...] = jnp.reshape(jnp.sum(loss) / B, (1, 1))


def kernel(inputs, targets):
    xt = inputs.T
    t2 = targets.reshape(1, B)
    out = pl.pallas_call(
        _loss_body,
        grid=(NV,),
        in_specs=[
            pl.BlockSpec((1, B), lambda v: (0, 0)),
            pl.BlockSpec((BV, B), lambda v: (v, 0)),
        ],
        out_specs=pl.BlockSpec((1, 1), lambda v: (0, 0)),
        out_shape=jax.ShapeDtypeStruct((1, 1), jnp.float32),
        scratch_shapes=[pltpu.VMEM((8, B), jnp.float32) for _ in range(4)],
    )(t2, xt)
    return out[0, 0]


# speedup vs baseline: 1.0083x; 1.0083x over previous
import jax
import jax.numpy as jnp
from jax.experimental import pallas as pl
from jax.experimental.pallas import tpu as pltpu

C = 100000
B = 1024
SMOOTH = 0.1
CONF = 1.0 - SMOOTH
SVAL = SMOOTH / (C - 1)

BV = 2000
NV = C // BV
NG = BV // 8
BVH = BV // 2
NGH = BVH // 8


def _loss_body(t_ref, xa_ref, xb_ref, o_ref, m_ref, s_ref, rs_ref, tg_ref):
    v = pl.program_id(0)
    nv = pl.num_programs(0)

    @pl.when(v == 0)
    def _init():
        m_ref[...] = jnp.full((8, B), -jnp.inf, jnp.float32)
        s_ref[...] = jnp.zeros((8, B), jnp.float32)
        rs_ref[...] = jnp.zeros((8, B), jnp.float32)
        tg_ref[...] = jnp.zeros((8, B), jnp.float32)

    sub = jax.lax.broadcasted_iota(jnp.int32, (8, B), 0)
    trel = t_ref[...] - v * BV + jnp.zeros((8, B), jnp.int32)

    def p1(ref):
        def body(g, carry):
            bmax, rs = carry
            xg = ref[pl.ds(pl.multiple_of(g * 8, 8), 8), :]
            return jnp.maximum(bmax, xg), rs + xg
        return body

    bmax, rs = jax.lax.fori_loop(
        0, NGH, p1(xa_ref),
        (jnp.full((8, B), -jnp.inf, jnp.float32), rs_ref[...]))
    bmax, rs = jax.lax.fori_loop(0, NGH, p1(xb_ref), (bmax, rs))
    rs_ref[...] = rs
    m_old = m_ref[...]
    m_new = jnp.maximum(m_old, bmax)
    m_ref[...] = m_new

    def p2(ref, goff):
        def body(g, carry):
            acc, tg = carry
            xg = ref[pl.ds(pl.multiple_of(g * 8, 8), 8), :]
            hit = trel - (g + goff) * 8 == sub
            return acc + jnp.exp(xg - m_new), tg + jnp.where(hit, xg, 0.0)
        return body

    acc, tg = jax.lax.fori_loop(
        0, NGH, p2(xa_ref, 0),
        (s_ref[...] * jnp.exp(m_old - m_new), tg_ref[...]))
    acc, tg = jax.lax.fori_loop(0, NGH, p2(xb_ref, NGH), (acc, tg))
    s_ref[...] = acc
    tg_ref[...] = tg

    @pl.when(v == nv - 1)
    def _fin():
        m_acc = m_ref[...]
        m_col = jnp.max(m_acc, axis=0, keepdims=True)
        s_col = jnp.sum(s_ref[...] * jnp.exp(m_acc - m_col), axis=0,
                        keepdims=True)
        lse = m_col + jnp.log(s_col)
        rs_col = jnp.sum(rs_ref[...], axis=0, keepdims=True)
        g_col = jnp.sum(tg_ref[...], axis=0, keepdims=True)
        loss = -(SVAL * (rs_col - C * lse) + (CONF - SVAL) * (g_col - lse))
        o_ref[...] = jnp.reshape(jnp.sum(loss) / B, (1, 1))


def kernel(inputs, targets):
    xt = inputs.T
    t2 = targets.reshape(1, B)
    out = pl.pallas_call(
        _loss_body,
        grid=(NV,),
        in_specs=[
            pl.BlockSpec((1, B), lambda v: (0, 0)),
            pl.BlockSpec((BVH, B), lambda v: (2 * v, 0)),
            pl.BlockSpec((BVH, B), lambda v: (2 * v + 1, 0)),
        ],
        out_specs=pl.BlockSpec((1, 1), lambda v: (0, 0)),
        out_shape=jax.ShapeDtypeStruct((1, 1), jnp.float32),
        scratch_shapes=[pltpu.VMEM((8, B), jnp.float32) for _ in range(4)],
    )(t2, xt, xt)
    return out[0, 0]


# speedup vs baseline: 1.1865x; 1.1767x over previous
import functools

import jax
import jax.numpy as jnp
from jax import lax
from jax.experimental import pallas as pl
from jax.experimental.pallas import tpu as pltpu
from jax.experimental.pallas import tpu_sc as plsc

C = 100000
B = 1024
SMOOTH = 0.1
CONF = 1.0 - SMOOTH
SVAL = SMOOTH / (C - 1)

C_TC = 60000
C_SC = C - C_TC

BV = 2000
NV = C_TC // BV
NG = BV // 8

SC_NW = 32
NWC = 8
NWR = 4
RPW = C_SC // NWR
CH = 400
NCH = RPW // CH


def _loss_body(t_ref, x_ref, o_ref, m_ref, s_ref, rs_ref, tg_ref):
    v = pl.program_id(0)
    nv = pl.num_programs(0)

    @pl.when(v == 0)
    def _init():
        m_ref[...] = jnp.full((8, B), -jnp.inf, jnp.float32)
        s_ref[...] = jnp.zeros((8, B), jnp.float32)
        rs_ref[...] = jnp.zeros((8, B), jnp.float32)
        tg_ref[...] = jnp.zeros((8, B), jnp.float32)

    sub = jax.lax.broadcasted_iota(jnp.int32, (8, B), 0)
    trel = t_ref[...] - v * BV + jnp.zeros((8, B), jnp.int32)

    def p1(g, carry):
        bmax, rs = carry
        xg = x_ref[pl.ds(pl.multiple_of(g * 8, 8), 8), :]
        return jnp.maximum(bmax, xg), rs + xg

    bmax, rs = jax.lax.fori_loop(
        0, NG, p1,
        (jnp.full((8, B), -jnp.inf, jnp.float32), rs_ref[...]))
    rs_ref[...] = rs
    m_old = m_ref[...]
    m_new = jnp.maximum(m_old, bmax)
    m_ref[...] = m_new

    def p2(g, carry):
        acc, tg = carry
        xg = x_ref[pl.ds(pl.multiple_of(g * 8, 8), 8), :]
        hit = trel - g * 8 == sub
        return acc + jnp.exp(xg - m_new), tg + jnp.where(hit, xg, 0.0)

    acc, tg = jax.lax.fori_loop(
        0, NG, p2, (s_ref[...] * jnp.exp(m_old - m_new), tg_ref[...]))
    s_ref[...] = acc
    tg_ref[...] = tg

    @pl.when(v == nv - 1)
    def _fin():
        o_ref[0:8, :] = m_ref[...]
        o_ref[8:16, :] = s_ref[...]
        o_ref[16:24, :] = rs_ref[...]
        o_ref[24:32, :] = tg_ref[...]


def _tc_partials(t2, xt):
    return pl.pallas_call(
        _loss_body,
        grid=(NV,),
        in_specs=[
            pl.BlockSpec((1, B), lambda v: (0, 0)),
            pl.BlockSpec((BV, B), lambda v: (v, 0)),
        ],
        out_specs=pl.BlockSpec((32, B), lambda v: (0, 0)),
        out_shape=jax.ShapeDtypeStruct((32, B), jnp.float32),
        scratch_shapes=[pltpu.VMEM((8, B), jnp.float32) for _ in range(4)],
    )(t2, xt)


def _sc_body(x_hbm, t_hbm, out_hbm, t_v, buf, stage, sem):
    w = lax.axis_index("s") * 2 + lax.axis_index("c")
    wc = w // NWR
    wr = w % NWR
    c0 = wc * 128
    pltpu.sync_copy(t_hbm.at[pl.ds(c0, 128)], t_v)
    iota = lax.iota(jnp.int32, 16)
    ts = [t_v[pl.ds(16 * c, 16)] for c in range(8)]
    ninf = jnp.full((16,), -jnp.inf, jnp.float32)
    zero = jnp.zeros((16,), jnp.float32)

    def chunk(i, carry):
        m = list(carry[0:8])
        s = list(carry[8:16])
        rs = list(carry[16:24])
        tg = list(carry[24:32])
        r0 = C_TC + wr * RPW + i * CH
        pltpu.async_copy(
            x_hbm.at[pl.ds(r0, CH), pl.ds(c0, 128)], buf, sem).wait()
        rel = [ts[c] - r0 for c in range(8)]

        def p1(r, c24):
            bm = list(c24[0:8])
            rl = list(c24[8:16])
            tl = list(c24[16:24])
            ri = jnp.full((16,), r, jnp.int32)
            for c in range(8):
                xc = buf[r, pl.ds(16 * c, 16)]
                bm[c] = jnp.maximum(bm[c], xc)
                rl[c] = rl[c] + xc
                tl[c] = tl[c] + jnp.where(rel[c] == ri, xc, 0.0)
            return tuple(bm) + tuple(rl) + tuple(tl)

        pc = lax.fori_loop(0, CH, p1, (ninf,) * 8 + (zero,) * 16)
        m2 = [jnp.maximum(m[c], pc[c]) for c in range(8)]
        s = [s[c] * jnp.exp(m[c] - m2[c]) for c in range(8)]

        def p2(r, c8):
            sl = list(c8)
            for c in range(8):
                sl[c] = sl[c] + jnp.exp(buf[r, pl.ds(16 * c, 16)] - m2[c])
            return tuple(sl)

        s = list(lax.fori_loop(0, CH, p2, tuple(s)))
        rs = [rs[c] + pc[8 + c] for c in range(8)]
        tg = [tg[c] + pc[16 + c] for c in range(8)]
        return tuple(m2) + tuple(s) + tuple(rs) + tuple(tg)

    res = lax.fori_loop(0, NCH, chunk, (ninf,) * 8 + (zero,) * 24)
    for k in range(4):
        for c in range(8):
            stage[pl.ds(k * 128 + 16 * c, 16)] = res[8 * k + c]
    for k in range(4):
        pltpu.sync_copy(
            stage.at[pl.ds(k * 128, 128)],
            out_hbm.at[pl.ds(k * NWR * B + wr * B + c0, 128)])


def _sc_partials(xt, targets):
    k = functools.partial(
        pl.kernel,
        mesh=plsc.VectorSubcoreMesh(core_axis_name="c", subcore_axis_name="s"),
        out_type=jax.ShapeDtypeStruct((4 * NWR * B,), jnp.float32),
        scratch_types=[
            pltpu.VMEM((128,), jnp.int32),
            pltpu.VMEM((CH, 128), jnp.float32),
            pltpu.VMEM((512,), jnp.float32),
            pltpu.SemaphoreType.DMA,
        ],
    )(_sc_body)
    return k(xt, targets)


def _merge_body(tc_ref, sc_ref, o_ref):
    tcp = tc_ref[...]
    scp = sc_ref[...]
    m8 = tcp[0:8, :]
    s8 = tcp[8:16, :]
    rs8 = tcp[16:24, :]
    tg8 = tcp[24:32, :]
    m_sc = scp[0:NWR, :]
    s_sc = scp[NWR:2 * NWR, :]
    rs_sc = scp[2 * NWR:3 * NWR, :]
    tg_sc = scp[3 * NWR:4 * NWR, :]
    m_col = jnp.maximum(jnp.max(m8, axis=0, keepdims=True),
                        jnp.max(m_sc, axis=0, keepdims=True))
    s_col = (jnp.sum(s8 * jnp.exp(m8 - m_col), axis=0, keepdims=True)
             + jnp.sum(s_sc * jnp.exp(m_sc - m_col), axis=0, keepdims=True))
    lse = m_col + jnp.log(s_col)
    rs_col = (jnp.sum(rs8, axis=0, keepdims=True)
              + jnp.sum(rs_sc, axis=0, keepdims=True))
    g_col = (jnp.sum(tg8, axis=0, keepdims=True)
             + jnp.sum(tg_sc, axis=0, keepdims=True))
    loss = -(SVAL * (rs_col - C * lse) + (CONF - SVAL) * (g_col - lse))
    o_ref[...] = jnp.reshape(jnp.sum(loss) / B, (1, 1))


def _merge(tcp, scp):
    return pl.pallas_call(
        _merge_body,
        in_specs=[
            pl.BlockSpec((32, B), lambda: (0, 0)),
            pl.BlockSpec((4 * NWR, B), lambda: (0, 0)),
        ],
        out_specs=pl.BlockSpec((1, 1), lambda: (0, 0)),
        out_shape=jax.ShapeDtypeStruct((1, 1), jnp.float32),
    )(tcp, scp)


def kernel(inputs, targets):
    xt = inputs.T
    t2 = targets.reshape(1, B)
    tcp = _tc_partials(t2, xt)
    scp = _sc_partials(xt, targets).reshape(4 * NWR, B)
    out = _merge(tcp, scp)
    return out[0, 0]


# speedup vs baseline: 1.3031x; 1.0982x over previous
import functools

import jax
import jax.numpy as jnp
from jax import lax
from jax.experimental import pallas as pl
from jax.experimental.pallas import tpu as pltpu
from jax.experimental.pallas import tpu_sc as plsc

C = 100000
B = 1024
SMOOTH = 0.1
CONF = 1.0 - SMOOTH
SVAL = SMOOTH / (C - 1)

C_TC = 64800
C_SC = C - C_TC

BV = 1800
NV = C_TC // BV
NG = BV // 8

SC_NW = 32
NWC = 8
NWR = 4
RPW = C_SC // NWR
CH = 400
NCH = RPW // CH


def _loss_body(t_ref, x_ref, o_ref, m_ref, s_ref, rs_ref, tg_ref):
    v = pl.program_id(0)
    nv = pl.num_programs(0)

    @pl.when(v == 0)
    def _init():
        m_ref[...] = jnp.full((8, B), -jnp.inf, jnp.float32)
        s_ref[...] = jnp.zeros((8, B), jnp.float32)
        rs_ref[...] = jnp.zeros((8, B), jnp.float32)
        tg_ref[...] = jnp.zeros((8, B), jnp.float32)

    sub = jax.lax.broadcasted_iota(jnp.int32, (8, B), 0)
    trel = t_ref[...] - v * BV + jnp.zeros((8, B), jnp.int32)

    def p1(g, carry):
        bmax, rs = carry
        xg = x_ref[pl.ds(pl.multiple_of(g * 8, 8), 8), :]
        return jnp.maximum(bmax, xg), rs + xg

    bmax, rs = jax.lax.fori_loop(
        0, NG, p1,
        (jnp.full((8, B), -jnp.inf, jnp.float32), rs_ref[...]))
    rs_ref[...] = rs
    m_old = m_ref[...]
    m_new = jnp.maximum(m_old, bmax)
    m_ref[...] = m_new

    def p2(g, carry):
        acc, tg = carry
        xg = x_ref[pl.ds(pl.multiple_of(g * 8, 8), 8), :]
        hit = trel - g * 8 == sub
        return acc + jnp.exp(xg - m_new), tg + jnp.where(hit, xg, 0.0)

    acc, tg = jax.lax.fori_loop(
        0, NG, p2, (s_ref[...] * jnp.exp(m_old - m_new), tg_ref[...]))
    s_ref[...] = acc
    tg_ref[...] = tg

    @pl.when(v == nv - 1)
    def _fin():
        o_ref[0:8, :] = m_ref[...]
        o_ref[8:16, :] = s_ref[...]
        o_ref[16:24, :] = rs_ref[...]
        o_ref[24:32, :] = tg_ref[...]


def _tc_partials(t2, xt):
    return pl.pallas_call(
        _loss_body,
        grid=(NV,),
        in_specs=[
            pl.BlockSpec((1, B), lambda v: (0, 0)),
            pl.BlockSpec((BV, B), lambda v: (v, 0)),
        ],
        out_specs=pl.BlockSpec((32, B), lambda v: (0, 0)),
        out_shape=jax.ShapeDtypeStruct((32, B), jnp.float32),
        scratch_shapes=[pltpu.VMEM((8, B), jnp.float32) for _ in range(4)],
    )(t2, xt)


def _sc_body(x_hbm, t_hbm, out_hbm, t_v, buf, stage, sem):
    w = lax.axis_index("s") * 2 + lax.axis_index("c")
    wc = w // NWR
    wr = w % NWR
    c0 = wc * 128
    pltpu.sync_copy(t_hbm.at[pl.ds(c0, 128)], t_v)
    iota = lax.iota(jnp.int32, 16)
    ts = [t_v[pl.ds(16 * c, 16)] for c in range(8)]
    ninf = jnp.full((16,), -jnp.inf, jnp.float32)
    zero = jnp.zeros((16,), jnp.float32)

    def chunk(i, carry):
        m = list(carry[0:8])
        s = list(carry[8:16])
        rs = list(carry[16:24])
        tg = list(carry[24:32])
        r0 = C_TC + wr * RPW + i * CH
        pltpu.async_copy(
            x_hbm.at[pl.ds(r0, CH), pl.ds(c0, 128)], buf, sem).wait()
        rel = [ts[c] - r0 for c in range(8)]

        def p1(r, c24):
            bm = list(c24[0:8])
            rl = list(c24[8:16])
            tl = list(c24[16:24])
            ri = jnp.full((16,), r, jnp.int32)
            for c in range(8):
                xc = buf[r, pl.ds(16 * c, 16)]
                bm[c] = jnp.maximum(bm[c], xc)
                rl[c] = rl[c] + xc
                tl[c] = tl[c] + jnp.where(rel[c] == ri, xc, 0.0)
            return tuple(bm) + tuple(rl) + tuple(tl)

        pc = lax.fori_loop(0, CH, p1, (ninf,) * 8 + (zero,) * 16)
        m2 = [jnp.maximum(m[c], pc[c]) for c in range(8)]
        s = [s[c] * jnp.exp(m[c] - m2[c]) for c in range(8)]

        def p2(r, c8):
            sl = list(c8)
            for c in range(8):
                sl[c] = sl[c] + jnp.exp(buf[r, pl.ds(16 * c, 16)] - m2[c])
            return tuple(sl)

        s = list(lax.fori_loop(0, CH, p2, tuple(s)))
        rs = [rs[c] + pc[8 + c] for c in range(8)]
        tg = [tg[c] + pc[16 + c] for c in range(8)]
        return tuple(m2) + tuple(s) + tuple(rs) + tuple(tg)

    res = lax.fori_loop(0, NCH, chunk, (ninf,) * 8 + (zero,) * 24)
    for k in range(4):
        for c in range(8):
            stage[pl.ds(k * 128 + 16 * c, 16)] = res[8 * k + c]
    for k in range(4):
        pltpu.sync_copy(
            stage.at[pl.ds(k * 128, 128)],
            out_hbm.at[pl.ds(k * NWR * B + wr * B + c0, 128)])


def _sc_partials(xt, targets):
    k = functools.partial(
        pl.kernel,
        mesh=plsc.VectorSubcoreMesh(core_axis_name="c", subcore_axis_name="s"),
        out_type=jax.ShapeDtypeStruct((4 * NWR * B,), jnp.float32),
        scratch_types=[
            pltpu.VMEM((128,), jnp.int32),
            pltpu.VMEM((CH, 128), jnp.float32),
            pltpu.VMEM((512,), jnp.float32),
            pltpu.SemaphoreType.DMA,
        ],
    )(_sc_body)
    return k(xt, targets)


def _merge_body(tc_ref, sc_ref, o_ref):
    tcp = tc_ref[...]
    scp = sc_ref[...]
    m8 = tcp[0:8, :]
    s8 = tcp[8:16, :]
    rs8 = tcp[16:24, :]
    tg8 = tcp[24:32, :]
    m_sc = scp[0:NWR, :]
    s_sc = scp[NWR:2 * NWR, :]
    rs_sc = scp[2 * NWR:3 * NWR, :]
    tg_sc = scp[3 * NWR:4 * NWR, :]
    m_col = jnp.maximum(jnp.max(m8, axis=0, keepdims=True),
                        jnp.max(m_sc, axis=0, keepdims=True))
    s_col = (jnp.sum(s8 * jnp.exp(m8 - m_col), axis=0, keepdims=True)
             + jnp.sum(s_sc * jnp.exp(m_sc - m_col), axis=0, keepdims=True))
    lse = m_col + jnp.log(s_col)
    rs_col = (jnp.sum(rs8, axis=0, keepdims=True)
              + jnp.sum(rs_sc, axis=0, keepdims=True))
    g_col = (jnp.sum(tg8, axis=0, keepdims=True)
             + jnp.sum(tg_sc, axis=0, keepdims=True))
    loss = -(SVAL * (rs_col - C * lse) + (CONF - SVAL) * (g_col - lse))
    o_ref[...] = jnp.reshape(jnp.sum(loss) / B, (1, 1))


def _merge(tcp, scp):
    return pl.pallas_call(
        _merge_body,
        in_specs=[
            pl.BlockSpec((32, B), lambda: (0, 0)),
            pl.BlockSpec((4 * NWR, B), lambda: (0, 0)),
        ],
        out_specs=pl.BlockSpec((1, 1), lambda: (0, 0)),
        out_shape=jax.ShapeDtypeStruct((1, 1), jnp.float32),
    )(tcp, scp)


def kernel(inputs, targets):
    xt = inputs.T
    t2 = targets.reshape(1, B)
    tcp = _tc_partials(t2, xt)
    scp = _sc_partials(xt, targets).reshape(4 * NWR, B)
    out = _merge(tcp, scp)
    return out[0, 0]
